# baseline (speedup 1.0000x reference)
import functools

import jax
import jax.numpy as jnp
from jax import lax
from jax.experimental import pallas as pl
from jax.experimental.pallas import tpu as pltpu
from jax.experimental.pallas import tpu_sc as plsc

N = 10000
E = 320000
D_IN = 128
D_HID = 128
D_OUT = 40
D_OP = 48
DH2 = 64
N_TGT = 1000
N_TGT_PAD = 1024

NC = 2
NS = 16
NW = NC * NS
N_PAD = 10240
E_PER_TILE = 10240
E_PAD = E_PER_TILE * NW
EB = 40
NB = E_PER_TILE // EB
EROWS = E_PAD // EB

NB_T = EROWS // NS
CH = 64
NCHUNK = NB_T // CH
J4C = CH // 4
NRING = 4

_MESH = plsc.VectorSubcoreMesh(core_axis_name="c", subcore_axis_name="s")
_NO_TC_TILING = pltpu.CompilerParams(use_tc_tiling_on_sc=False,
                                     needs_layout_passes=False)


def _zero_rows(buf, nrows, ncol16):
    z = jnp.zeros((16,), jnp.float32)
    for i in range(nrows):
        for c in range(ncol16):
            buf[i, pl.ds(c * 16, 16)] = z


def _ew_vec(sb_ew, idxs, j16):
    cols = jnp.minimum(lax.iota(jnp.int32, 16) + j16 * 16, EB - 1)
    full = [jnp.full((16,), i, jnp.int32) for i in idxs]
    return plsc.load_gather(sb_ew, full + [cols])


@functools.partial(
    pl.kernel,
    out_type=jax.ShapeDtypeStruct((NC, N_PAD), jnp.float32),
    mesh=_MESH,
    compiler_params=_NO_TC_TILING,
    scratch_types=[
        pltpu.VMEM((NB, EB), jnp.int32),
        pltpu.VMEM((NB, EB), jnp.float32),
        pltpu.VMEM((640,), jnp.float32),
        pltpu.VMEM_SHARED((N_PAD,), jnp.float32),
        pltpu.SemaphoreType.DMA,
        pltpu.SemaphoreType.DMA,
    ],
)
def _sc_degree(dst_hbm, ew_hbm, out_hbm, sb_dst, sb_ew, zbuf, acc, ss0, ss1):
    cid = lax.axis_index("c")
    sid = lax.axis_index("s")
    tid = cid * NS + sid

    pltpu.sync_copy(dst_hbm.at[pl.ds(tid * NB, NB)], sb_dst)
    pltpu.sync_copy(ew_hbm.at[pl.ds(tid * NB, NB)], sb_ew)
    z = jnp.zeros((16,), jnp.float32)
    for i in range(40):
        zbuf[pl.ds(i * 16, 16)] = z
    pltpu.sync_copy(zbuf, acc.at[pl.ds(sid * 640, 640)])
    plsc.subcore_barrier()

    def body(j2, _):
        for par, ss in ((0, ss0), (1, ss1)):
            j = j2 * 2 + par

            @pl.when(j2 >= 1)
            def _():
                pltpu.make_async_copy(sb_ew.at[j], acc.at[sb_dst.at[j]],
                                      ss).wait()
            pltpu.async_copy(sb_ew.at[j], acc.at[sb_dst.at[j]], ss, add=True)
        return 0
    lax.fori_loop(0, NB // 2, body, 0)
    pltpu.make_async_copy(sb_ew.at[0], acc.at[sb_dst.at[0]], ss0).wait()
    pltpu.make_async_copy(sb_ew.at[0], acc.at[sb_dst.at[0]], ss1).wait()
    plsc.subcore_barrier()
    pltpu.sync_copy(acc.at[pl.ds(sid * 640, 640)],
                    out_hbm.at[cid, pl.ds(sid * 640, 640)])


@functools.partial(
    pl.kernel,
    out_type=jax.ShapeDtypeStruct((NC, N_PAD, DH2), jnp.float32),
    mesh=_MESH,
    compiler_params=_NO_TC_TILING,
    scratch_types=[
        pltpu.VMEM((2, CH, EB), jnp.int32),
        pltpu.VMEM((2, CH, EB), jnp.int32),
        pltpu.VMEM((2, CH, EB), jnp.float32),
    ] + [pltpu.VMEM((EB, DH2), jnp.float32)] * (2 * NRING)
      + [pltpu.SemaphoreType.DMA] * (2 * NRING + 1)
      + [pltpu.VMEM_SHARED((N_PAD, DH2), jnp.float32)],
)
def _sc_edge_feat(gf_hbm, src_hbm, dst_hbm, ew_hbm, out_hbm,
                  sb_src, sb_dst, sb_ew, *rest):
    grows = rest[0:NRING]
    srows = rest[NRING:2 * NRING]
    sgs = rest[2 * NRING:3 * NRING]
    sss = rest[3 * NRING:4 * NRING]
    sst = rest[4 * NRING]
    acc = rest[4 * NRING + 1]
    cid = lax.axis_index("c")
    sid = lax.axis_index("s")
    row0 = sid * NB_T

    def stage(chunk, p, sem):
        r = row0 + chunk * CH
        pltpu.async_copy(src_hbm.at[cid, pl.ds(r, CH)], sb_src.at[p], sem)
        pltpu.async_copy(dst_hbm.at[pl.ds(r, CH)], sb_dst.at[p], sem)
        pltpu.async_copy(ew_hbm.at[pl.ds(r, CH)], sb_ew.at[p], sem)

    def stage_wait(sem):
        pltpu.make_async_copy(src_hbm.at[cid, pl.ds(row0, CH)],
                              sb_src.at[0], sem).wait()
        pltpu.make_async_copy(dst_hbm.at[pl.ds(row0, CH)],
                              sb_dst.at[0], sem).wait()
        pltpu.make_async_copy(ew_hbm.at[pl.ds(row0, CH)],
                              sb_ew.at[0], sem).wait()

    stage(0, 0, sst)
    _zero_rows(srows[0], EB, DH2 // 16)
    for q in range(640 // EB):
        pltpu.sync_copy(srows[0], acc.at[pl.ds(sid * 640 + q * EB, EB)])
    stage_wait(sst)
    plsc.subcore_barrier()

    def gidx(j):
        return (j // CH) % 2, j % CH

    def scale(j, grow, srow):
        p, jw = gidx(j)
        for j16 in range(3):
            ewv = _ew_vec(sb_ew, (p, jw), j16)
            for l in range(16):
                r = j16 * 16 + l
                if r >= EB:
                    break
                s = ewv[l]
                for c in range(DH2 // 16):
                    srow[r, pl.ds(c * 16, 16)] = (
                        grow[r, pl.ds(c * 16, 16)] * s)

    def g_issue(j, grow, sg):
        p, jw = gidx(j)
        pltpu.async_copy(gf_hbm.at[sb_src.at[p, jw]], grow, sg)

    def g_wait(grow, sg):
        pltpu.make_async_copy(gf_hbm.at[sb_src.at[0, 0]], grow, sg).wait()

    def s_issue(j, srow, ss):
        p, jw = gidx(j)
        pltpu.async_copy(srow, acc.at[sb_dst.at[p, jw]], ss, add=True)

    def s_wait(srow, ss):
        pltpu.make_async_copy(srow, acc.at[sb_dst.at[0, 0]], ss).wait()

    for par in range(NRING):
        g_issue(par, grows[par], sgs[par])

    def body(j4, _):
        for par in range(NRING):
            grow, srow, sg, ss = grows[par], srows[par], sgs[par], sss[par]
            j = j4 * NRING + par
            g_wait(grow, sg)

            @pl.when(j4 >= 1)
            def _():
                s_wait(srow, ss)
            scale(j, grow, srow)
            s_issue(j, srow, ss)

            @pl.when(j4 < NB_T // NRING - 1)
            def _():
                if par == 0:
                    @pl.when(j4 % J4C == J4C - 1)
                    def _():
                        stage_wait(sst)
                g_issue(j + NRING, grow, sg)

            if par == NRING - 1:
                @pl.when((j4 % J4C == 0) & (j4 < (NCHUNK - 1) * J4C))
                def _():
                    stage(j4 // J4C + 1, (j4 // J4C + 1) % 2, sst)
        return 0
    lax.fori_loop(0, NB_T // NRING, body, 0)
    for par in range(NRING):
        s_wait(srows[par], sss[par])
    plsc.subcore_barrier()
    pltpu.sync_copy(acc.at[pl.ds(sid * 640, 640)],
                    out_hbm.at[cid, pl.ds(sid * 640, 640)])


@functools.partial(
    pl.kernel,
    out_type=jax.ShapeDtypeStruct((NC, N_PAD, D_OP), jnp.float32),
    mesh=_MESH,
    compiler_params=_NO_TC_TILING,
    scratch_types=[
        pltpu.VMEM((NB, EB), jnp.int32),
        pltpu.VMEM((NB, EB), jnp.int32),
        pltpu.VMEM((NB, EB), jnp.float32),
    ] + [pltpu.VMEM((EB, D_OP), jnp.float32)] * (2 * NRING)
      + [pltpu.SemaphoreType.DMA] * (2 * NRING)
      + [pltpu.VMEM_SHARED((N_PAD, D_OP), jnp.float32)],
)
def _sc_edge_out(g_hbm, src_hbm, dst_hbm, ew_hbm, out_hbm,
                 sb_src, sb_dst, sb_ew, *rest):
    grows = rest[0:NRING]
    srows = rest[NRING:2 * NRING]
    sgs = rest[2 * NRING:3 * NRING]
    sss = rest[3 * NRING:4 * NRING]
    acc = rest[4 * NRING]
    cid = lax.axis_index("c")
    sid = lax.axis_index("s")
    tid = cid * NS + sid

    pltpu.sync_copy(src_hbm.at[pl.ds(tid * NB, NB)], sb_src)
    pltpu.sync_copy(dst_hbm.at[pl.ds(tid * NB, NB)], sb_dst)
    pltpu.sync_copy(ew_hbm.at[pl.ds(tid * NB, NB)], sb_ew)

    _zero_rows(srows[0], EB, D_OP // 16)
    for q in range(640 // EB):
        pltpu.sync_copy(srows[0], acc.at[pl.ds(sid * 640 + q * EB, EB)])
    plsc.subcore_barrier()

    def scale(j, grow, srow):
        for j16 in range(3):
            ewv = _ew_vec(sb_ew, (j,), j16)
            for l in range(16):
                r = j16 * 16 + l
                if r >= EB:
                    break
                s = ewv[l]
                for c in range(D_OP // 16):
                    srow[r, pl.ds(c * 16, 16)] = (
                        grow[r, pl.ds(c * 16, 16)] * s)

    for par in range(NRING):
        pltpu.async_copy(g_hbm.at[sb_src.at[par]], grows[par], sgs[par])

    def body(j4, _):
        for par in range(NRING):
            grow, srow, sg, ss = grows[par], srows[par], sgs[par], sss[par]
            j = j4 * NRING + par
            pltpu.make_async_copy(g_hbm.at[sb_src.at[0]], grow, sg).wait()

            @pl.when(j4 >= 1)
            def _():
                pltpu.make_async_copy(srow, acc.at[sb_dst.at[0]], ss).wait()
            scale(j, grow, srow)
            pltpu.async_copy(srow, acc.at[sb_dst.at[j]], ss, add=True)

            @pl.when(j4 < NB // NRING - 1)
            def _():
                pltpu.async_copy(g_hbm.at[sb_src.at[j + NRING]], grow, sg)
        return 0
    lax.fori_loop(0, NB // NRING, body, 0)
    for par in range(NRING):
        pltpu.make_async_copy(srows[par], acc.at[sb_dst.at[0]], sss[par]).wait()
    plsc.subcore_barrier()
    pltpu.sync_copy(acc.at[pl.ds(sid * 640, 640)],
                    out_hbm.at[cid, pl.ds(sid * 640, 640)])


_TPW = N_TGT_PAD // NW


@functools.partial(
    pl.kernel,
    out_type=[jax.ShapeDtypeStruct((N_TGT_PAD, D_OP), jnp.float32),
              jax.ShapeDtypeStruct((N_TGT_PAD, 1), jnp.float32)],
    mesh=_MESH,
    compiler_params=_NO_TC_TILING,
    scratch_types=[
        pltpu.VMEM((_TPW,), jnp.int32),
        pltpu.VMEM((_TPW, D_OP), jnp.float32),
        pltpu.VMEM((_TPW, D_OP), jnp.float32),
        pltpu.VMEM((_TPW, D_OP), jnp.float32),
        pltpu.VMEM((_TPW, 1), jnp.float32),
        pltpu.SemaphoreType.DMA,
    ],
)
def _sc_target_gather(s2a_hbm, s2b_hbm, g2_hbm, dinv_hbm, tgt_hbm,
                      ypre_hbm, dinvt_hbm,
                      idx_t, rows_a, rows_b, rows_g, rows_d, sem):
    cid = lax.axis_index("c")
    sid = lax.axis_index("s")
    base = (cid * NS + sid) * _TPW
    b = pl.multiple_of(base, 8)
    pltpu.sync_copy(tgt_hbm.at[pl.ds(b, _TPW)], idx_t)
    pltpu.async_copy(s2a_hbm.at[idx_t], rows_a, sem)
    pltpu.async_copy(s2b_hbm.at[idx_t], rows_b, sem)
    pltpu.async_copy(g2_hbm.at[idx_t], rows_g, sem)
    pltpu.async_copy(dinv_hbm.at[idx_t], rows_d, sem)
    pltpu.make_async_copy(s2a_hbm.at[idx_t], rows_a, sem).wait()
    pltpu.make_async_copy(s2b_hbm.at[idx_t], rows_b, sem).wait()
    pltpu.make_async_copy(g2_hbm.at[idx_t], rows_g, sem).wait()
    pltpu.make_async_copy(dinv_hbm.at[idx_t], rows_d, sem).wait()
    for r in range(_TPW):
        for c in range(D_OP // 16):
            sl = pl.ds(c * 16, 16)
            rows_a[r, sl] = rows_a[r, sl] + rows_b[r, sl] + rows_g[r, sl]
    pltpu.sync_copy(rows_a, ypre_hbm.at[pl.ds(b, _TPW)])
    pltpu.sync_copy(rows_d, dinvt_hbm.at[pl.ds(b, _TPW)])


def _tc1_body(x_ref, w1_ref, deg_ref, g1f_ref):
    deg = deg_ref[0, :N] + deg_ref[1, :N] + 1.0
    dinv = lax.rsqrt(deg)
    h1 = jnp.dot(x_ref[...], w1_ref[...], preferred_element_type=jnp.float32)
    g1 = dinv[:, None] * h1
    g1f_ref[:N, :] = g1[:, :DH2]
    g1f_ref[N:, :] = g1[:, DH2:]


def _tc2_body(s1_ref, g1f_ref, deg_ref, b1_ref, w2_ref, g2_ref, dinv_ref):
    deg = deg_ref[0, :N] + deg_ref[1, :N] + 1.0
    dinv = lax.rsqrt(deg)
    s1 = jnp.concatenate([s1_ref[0, :N, :], s1_ref[1, :N, :]], axis=1)
    g1 = jnp.concatenate([g1f_ref[:N, :], g1f_ref[N:, :]], axis=1)
    out1 = dinv[:, None] * (s1 + g1) + b1_ref[...]
    h2 = jnp.maximum(out1, 0.0)
    f2 = jnp.dot(h2, w2_ref[...], preferred_element_type=jnp.float32)
    g2_ref[...] = dinv[:, None] * f2
    dinv_ref[...] = dinv[:, None]


def _tc4_body(ypre_ref, dinvt_ref, b2_ref, tgt_ref, loss_ref, y_ref):
    yv = (dinvt_ref[:N_TGT] * ypre_ref[:N_TGT, :D_OUT]
          + b2_ref[:, :D_OUT])
    m = jnp.max(yv, axis=1, keepdims=True)
    ex = jnp.exp(yv - m)
    lse = m[:, 0] + jnp.log(jnp.sum(ex, axis=1))
    cls = lax.broadcasted_iota(jnp.int32, (N_TGT, D_OUT), 1)
    picked = jnp.sum(jnp.where(cls == tgt_ref[...], yv, 0.0), axis=1)
    loss_ref[...] = jnp.mean(lse - picked).reshape(1, 1)
    y_ref[...] = yv


def kernel(x, edge_index, edge_weight, target_x, target, W1, b1, W2, b2):
    pad_idx = jnp.arange(E_PAD - E, dtype=jnp.int32) % N
    src = jnp.concatenate(
        [edge_index[0].astype(jnp.int32), pad_idx]).reshape(EROWS, EB)
    dst = jnp.concatenate(
        [edge_index[1].astype(jnp.int32), pad_idx]).reshape(EROWS, EB)
    ew = jnp.concatenate(
        [edge_weight.astype(jnp.float32),
         jnp.zeros((E_PAD - E,), jnp.float32)]).reshape(EROWS, EB)
    src2 = jnp.stack([src, src + N])
    tgt_pad = jnp.concatenate(
        [target_x.astype(jnp.int32),
         jnp.zeros((N_TGT_PAD - N_TGT,), jnp.int32)])
    W2p = jnp.pad(W2, ((0, 0), (0, D_OP - D_OUT)))
    b2p = jnp.pad(b2, (0, D_OP - D_OUT))

    deg_parts = _sc_degree(dst, ew)

    g1f = pl.pallas_call(
        _tc1_body,
        out_shape=jax.ShapeDtypeStruct((2 * N, DH2), jnp.float32),
    )(x, W1, deg_parts)

    s1_parts = _sc_edge_feat(g1f, src2, dst, ew)

    g2, dinv_col = pl.pallas_call(
        _tc2_body,
        out_shape=[
            jax.ShapeDtypeStruct((N, D_OP), jnp.float32),
            jax.ShapeDtypeStruct((N, 1), jnp.float32),
        ],
    )(s1_parts, g1f, deg_parts, b1.reshape(1, D_HID), W2p)

    s2_parts = _sc_edge_out(g2, src, dst, ew)

    ypre, dinvt = _sc_target_gather(
        s2_parts[0], s2_parts[1], g2, dinv_col, tgt_pad)

    loss_arr, y = pl.pallas_call(
        _tc4_body,
        out_shape=[
            jax.ShapeDtypeStruct((1, 1), jnp.float32),
            jax.ShapeDtypeStruct((N_TGT, D_OUT), jnp.float32),
        ],
    )(ypre, dinvt, b2p.reshape(1, D_OP),
      target.astype(jnp.int32).reshape(N_TGT, 1))

    return (loss_arr[0, 0], y)


# speedup vs baseline: 1.0359x; 1.0359x over previous
import functools

import jax
import jax.numpy as jnp
from jax import lax
from jax.experimental import pallas as pl
from jax.experimental.pallas import tpu as pltpu
from jax.experimental.pallas import tpu_sc as plsc

N = 10000
E = 320000
D_IN = 128
D_HID = 128
D_OUT = 40
D_OP = 48
DH2 = 64
N_TGT = 1000
N_TGT_PAD = 1024

NC = 2
NS = 16
NW = NC * NS
N_PAD = 10240
E_PER_TILE = E // NW
EB = 40
NB = E_PER_TILE // EB
EROWS = E // EB

NB_T = EROWS // NS
CH = 20
NCHUNK = NB_T // CH
J4C = CH // 4
NRING = 4

_MESH = plsc.VectorSubcoreMesh(core_axis_name="c", subcore_axis_name="s")
_NO_TC_TILING = pltpu.CompilerParams(use_tc_tiling_on_sc=False,
                                     needs_layout_passes=False)


def _zero_rows(buf, nrows, ncol16):
    z = jnp.zeros((16,), jnp.float32)
    for i in range(nrows):
        for c in range(ncol16):
            buf[i, pl.ds(c * 16, 16)] = z


def _ew_vec(sb_ew, idxs, j16):
    cols = jnp.minimum(lax.iota(jnp.int32, 16) + j16 * 16, EB - 1)
    full = [jnp.full((16,), i, jnp.int32) for i in idxs]
    return plsc.load_gather(sb_ew, full + [cols])


@functools.partial(
    pl.kernel,
    out_type=jax.ShapeDtypeStruct((NC, N_PAD), jnp.float32),
    mesh=_MESH,
    compiler_params=_NO_TC_TILING,
    scratch_types=[
        pltpu.VMEM((NB, EB), jnp.int32),
        pltpu.VMEM((NB, EB), jnp.float32),
        pltpu.VMEM((640,), jnp.float32),
        pltpu.VMEM_SHARED((N_PAD,), jnp.float32),
        pltpu.SemaphoreType.DMA,
        pltpu.SemaphoreType.DMA,
    ],
)
def _sc_degree(dst_hbm, ew_hbm, out_hbm, sb_dst, sb_ew, zbuf, acc, ss0, ss1):
    cid = lax.axis_index("c")
    sid = lax.axis_index("s")
    tid = cid * NS + sid

    pltpu.sync_copy(dst_hbm.at[pl.ds(tid * NB, NB)], sb_dst)
    pltpu.sync_copy(ew_hbm.at[pl.ds(tid * NB, NB)], sb_ew)
    z = jnp.zeros((16,), jnp.float32)
    for i in range(40):
        zbuf[pl.ds(i * 16, 16)] = z
    pltpu.sync_copy(zbuf, acc.at[pl.ds(sid * 640, 640)])
    plsc.subcore_barrier()

    def body(j2, _):
        for par, ss in ((0, ss0), (1, ss1)):
            j = j2 * 2 + par

            @pl.when(j2 >= 1)
            def _():
                pltpu.make_async_copy(sb_ew.at[j], acc.at[sb_dst.at[j]],
                                      ss).wait()
            pltpu.async_copy(sb_ew.at[j], acc.at[sb_dst.at[j]], ss, add=True)
        return 0
    lax.fori_loop(0, NB // 2, body, 0)
    pltpu.make_async_copy(sb_ew.at[0], acc.at[sb_dst.at[0]], ss0).wait()
    pltpu.make_async_copy(sb_ew.at[0], acc.at[sb_dst.at[0]], ss1).wait()
    plsc.subcore_barrier()
    pltpu.sync_copy(acc.at[pl.ds(sid * 640, 640)],
                    out_hbm.at[cid, pl.ds(sid * 640, 640)])


@functools.partial(
    pl.kernel,
    out_type=jax.ShapeDtypeStruct((NC, N_PAD, DH2), jnp.float32),
    mesh=_MESH,
    compiler_params=_NO_TC_TILING,
    scratch_types=[
        pltpu.VMEM((2, CH, EB), jnp.int32),
        pltpu.VMEM((2, CH, EB), jnp.int32),
        pltpu.VMEM((2, CH, EB), jnp.float32),
    ] + [pltpu.VMEM((EB, DH2), jnp.float32)] * (2 * NRING)
      + [pltpu.SemaphoreType.DMA] * (2 * NRING + 1)
      + [pltpu.VMEM_SHARED((N_PAD, DH2), jnp.float32)],
)
def _sc_edge_feat(gf_hbm, src_hbm, dst_hbm, ew_hbm, out_hbm,
                  sb_src, sb_dst, sb_ew, *rest):
    grows = rest[0:NRING]
    srows = rest[NRING:2 * NRING]
    sgs = rest[2 * NRING:3 * NRING]
    sss = rest[3 * NRING:4 * NRING]
    sst = rest[4 * NRING]
    acc = rest[4 * NRING + 1]
    cid = lax.axis_index("c")
    sid = lax.axis_index("s")
    row0 = sid * NB_T

    def stage(chunk, p, sem):
        r = row0 + chunk * CH
        pltpu.async_copy(src_hbm.at[cid, pl.ds(r, CH)], sb_src.at[p], sem)
        pltpu.async_copy(dst_hbm.at[pl.ds(r, CH)], sb_dst.at[p], sem)
        pltpu.async_copy(ew_hbm.at[pl.ds(r, CH)], sb_ew.at[p], sem)

    def stage_wait(sem):
        pltpu.make_async_copy(src_hbm.at[cid, pl.ds(row0, CH)],
                              sb_src.at[0], sem).wait()
        pltpu.make_async_copy(dst_hbm.at[pl.ds(row0, CH)],
                              sb_dst.at[0], sem).wait()
        pltpu.make_async_copy(ew_hbm.at[pl.ds(row0, CH)],
                              sb_ew.at[0], sem).wait()

    stage(0, 0, sst)
    _zero_rows(srows[0], EB, DH2 // 16)
    for q in range(640 // EB):
        pltpu.sync_copy(srows[0], acc.at[pl.ds(sid * 640 + q * EB, EB)])
    stage_wait(sst)
    plsc.subcore_barrier()

    def gidx(j):
        return (j // CH) % 2, j % CH

    def scale(j, grow, srow):
        p, jw = gidx(j)
        for j16 in range(3):
            ewv = _ew_vec(sb_ew, (p, jw), j16)
            for l in range(16):
                r = j16 * 16 + l
                if r >= EB:
                    break
                s = ewv[l]
                for c in range(DH2 // 16):
                    srow[r, pl.ds(c * 16, 16)] = (
                        grow[r, pl.ds(c * 16, 16)] * s)

    def g_issue(j, grow, sg):
        p, jw = gidx(j)
        pltpu.async_copy(gf_hbm.at[sb_src.at[p, jw]], grow, sg)

    def g_wait(grow, sg):
        pltpu.make_async_copy(gf_hbm.at[sb_src.at[0, 0]], grow, sg).wait()

    def s_issue(j, srow, ss):
        p, jw = gidx(j)
        pltpu.async_copy(srow, acc.at[sb_dst.at[p, jw]], ss, add=True)

    def s_wait(srow, ss):
        pltpu.make_async_copy(srow, acc.at[sb_dst.at[0, 0]], ss).wait()

    for par in range(NRING):
        g_issue(par, grows[par], sgs[par])

    def body(j4, _):
        for par in range(NRING):
            grow, srow, sg, ss = grows[par], srows[par], sgs[par], sss[par]
            j = j4 * NRING + par
            g_wait(grow, sg)

            @pl.when(j4 >= 1)
            def _():
                s_wait(srow, ss)
            scale(j, grow, srow)
            s_issue(j, srow, ss)

            @pl.when(j4 < NB_T // NRING - 1)
            def _():
                if par == 0:
                    @pl.when(j4 % J4C == J4C - 1)
                    def _():
                        stage_wait(sst)
                g_issue(j + NRING, grow, sg)

            if par == NRING - 1:
                @pl.when((j4 % J4C == 0) & (j4 < (NCHUNK - 1) * J4C))
                def _():
                    stage(j4 // J4C + 1, (j4 // J4C + 1) % 2, sst)
        return 0
    lax.fori_loop(0, NB_T // NRING, body, 0)
    for par in range(NRING):
        s_wait(srows[par], sss[par])
    plsc.subcore_barrier()
    pltpu.sync_copy(acc.at[pl.ds(sid * 640, 640)],
                    out_hbm.at[cid, pl.ds(sid * 640, 640)])


@functools.partial(
    pl.kernel,
    out_type=jax.ShapeDtypeStruct((NC, N_PAD, D_OP), jnp.float32),
    mesh=_MESH,
    compiler_params=_NO_TC_TILING,
    scratch_types=[
        pltpu.VMEM((NB, EB), jnp.int32),
        pltpu.VMEM((NB, EB), jnp.int32),
        pltpu.VMEM((NB, EB), jnp.float32),
    ] + [pltpu.VMEM((EB, D_OP), jnp.float32)] * (2 * NRING)
      + [pltpu.SemaphoreType.DMA] * (2 * NRING)
      + [pltpu.VMEM_SHARED((N_PAD, D_OP), jnp.float32)],
)
def _sc_edge_out(g_hbm, src_hbm, dst_hbm, ew_hbm, out_hbm,
                 sb_src, sb_dst, sb_ew, *rest):
    grows = rest[0:NRING]
    srows = rest[NRING:2 * NRING]
    sgs = rest[2 * NRING:3 * NRING]
    sss = rest[3 * NRING:4 * NRING]
    acc = rest[4 * NRING]
    cid = lax.axis_index("c")
    sid = lax.axis_index("s")
    tid = cid * NS + sid

    pltpu.sync_copy(src_hbm.at[pl.ds(tid * NB, NB)], sb_src)
    pltpu.sync_copy(dst_hbm.at[pl.ds(tid * NB, NB)], sb_dst)
    pltpu.sync_copy(ew_hbm.at[pl.ds(tid * NB, NB)], sb_ew)

    _zero_rows(srows[0], EB, D_OP // 16)
    for q in range(640 // EB):
        pltpu.sync_copy(srows[0], acc.at[pl.ds(sid * 640 + q * EB, EB)])
    plsc.subcore_barrier()

    def scale(j, grow, srow):
        for j16 in range(3):
            ewv = _ew_vec(sb_ew, (j,), j16)
            for l in range(16):
                r = j16 * 16 + l
                if r >= EB:
                    break
                s = ewv[l]
                for c in range(D_OP // 16):
                    srow[r, pl.ds(c * 16, 16)] = (
                        grow[r, pl.ds(c * 16, 16)] * s)

    for par in range(NRING):
        pltpu.async_copy(g_hbm.at[sb_src.at[par]], grows[par], sgs[par])

    def body(j4, _):
        for par in range(NRING):
            grow, srow, sg, ss = grows[par], srows[par], sgs[par], sss[par]
            j = j4 * NRING + par
            pltpu.make_async_copy(g_hbm.at[sb_src.at[0]], grow, sg).wait()

            @pl.when(j4 >= 1)
            def _():
                pltpu.make_async_copy(srow, acc.at[sb_dst.at[0]], ss).wait()
            scale(j, grow, srow)
            pltpu.async_copy(srow, acc.at[sb_dst.at[j]], ss, add=True)

            @pl.when(j + NRING < NB)
            def _():
                pltpu.async_copy(g_hbm.at[sb_src.at[j + NRING]], grow, sg)
        return 0
    lax.fori_loop(0, NB // NRING, body, 0)
    for par in range(NB - (NB // NRING) * NRING):
        j = (NB // NRING) * NRING + par
        grow, srow, sg, ss = grows[par], srows[par], sgs[par], sss[par]
        pltpu.make_async_copy(g_hbm.at[sb_src.at[0]], grow, sg).wait()
        pltpu.make_async_copy(srow, acc.at[sb_dst.at[0]], ss).wait()
        scale(j, grow, srow)
        pltpu.async_copy(srow, acc.at[sb_dst.at[j]], ss, add=True)
    for par in range(NRING):
        pltpu.make_async_copy(srows[par], acc.at[sb_dst.at[0]], sss[par]).wait()
    plsc.subcore_barrier()
    pltpu.sync_copy(acc.at[pl.ds(sid * 640, 640)],
                    out_hbm.at[cid, pl.ds(sid * 640, 640)])


_TPW = N_TGT_PAD // NW


@functools.partial(
    pl.kernel,
    out_type=jax.ShapeDtypeStruct((N_TGT_PAD, D_OP), jnp.float32),
    mesh=_MESH,
    compiler_params=_NO_TC_TILING,
    scratch_types=[
        pltpu.VMEM((_TPW,), jnp.int32),
        pltpu.VMEM((_TPW, D_OP), jnp.float32),
        pltpu.SemaphoreType.DMA,
    ],
)
def _sc_target_gather(out2_hbm, tgt_hbm, y_hbm, idx_t, rows_t, sem):
    cid = lax.axis_index("c")
    sid = lax.axis_index("s")
    base = (cid * NS + sid) * _TPW
    b = pl.multiple_of(base, 8)
    pltpu.sync_copy(tgt_hbm.at[pl.ds(b, _TPW)], idx_t)
    pltpu.async_copy(out2_hbm.at[idx_t], rows_t, sem).wait()
    pltpu.sync_copy(rows_t, y_hbm.at[pl.ds(b, _TPW)])


def _tc1_body(x_ref, w1_ref, deg_ref, g1f_ref):
    deg = deg_ref[0, :N] + deg_ref[1, :N] + 1.0
    dinv = lax.rsqrt(deg)
    h1 = jnp.dot(x_ref[...], w1_ref[...], preferred_element_type=jnp.float32)
    g1 = dinv[:, None] * h1
    g1f_ref[:N, :] = g1[:, :DH2]
    g1f_ref[N:, :] = g1[:, DH2:]


def _tc2_body(s1_ref, g1f_ref, deg_ref, b1_ref, w2_ref, g2_ref):
    deg = deg_ref[0, :N] + deg_ref[1, :N] + 1.0
    dinv = lax.rsqrt(deg)
    s1 = jnp.concatenate([s1_ref[0, :N, :], s1_ref[1, :N, :]], axis=1)
    g1 = jnp.concatenate([g1f_ref[:N, :], g1f_ref[N:, :]], axis=1)
    out1 = dinv[:, None] * (s1 + g1) + b1_ref[...]
    h2 = jnp.maximum(out1, 0.0)
    f2 = jnp.dot(h2, w2_ref[...], preferred_element_type=jnp.float32)
    g2_ref[...] = dinv[:, None] * f2


def _tc3_body(s2_ref, g2_ref, deg_ref, b2_ref, out2_ref):
    deg = deg_ref[0, :N] + deg_ref[1, :N] + 1.0
    dinv = lax.rsqrt(deg)
    s2 = s2_ref[0, :N, :] + s2_ref[1, :N, :]
    out2_ref[...] = dinv[:, None] * (s2 + g2_ref[...]) + b2_ref[...]


def _tc4_body(y48_ref, tgt_ref, loss_ref, y_ref):
    yv = y48_ref[:N_TGT, :D_OUT]
    m = jnp.max(yv, axis=1, keepdims=True)
    ex = jnp.exp(yv - m)
    lse = m[:, 0] + jnp.log(jnp.sum(ex, axis=1))
    cls = lax.broadcasted_iota(jnp.int32, (N_TGT, D_OUT), 1)
    picked = jnp.sum(jnp.where(cls == tgt_ref[...], yv, 0.0), axis=1)
    loss_ref[...] = jnp.mean(lse - picked).reshape(1, 1)
    y_ref[...] = yv


def kernel(x, edge_index, edge_weight, target_x, target, W1, b1, W2, b2):
    src = edge_index[0].astype(jnp.int32).reshape(EROWS, EB)
    dst = edge_index[1].astype(jnp.int32).reshape(EROWS, EB)
    ew = edge_weight.astype(jnp.float32).reshape(EROWS, EB)
    src2 = jnp.stack([src, src + N])
    tgt_pad = jnp.concatenate(
        [target_x.astype(jnp.int32),
         jnp.zeros((N_TGT_PAD - N_TGT,), jnp.int32)])
    W2p = jnp.pad(W2, ((0, 0), (0, D_OP - D_OUT)))
    b2p = jnp.pad(b2, (0, D_OP - D_OUT))

    deg_parts = _sc_degree(dst, ew)

    g1f = pl.pallas_call(
        _tc1_body,
        out_shape=jax.ShapeDtypeStruct((2 * N, DH2), jnp.float32),
    )(x, W1, deg_parts)

    s1_parts = _sc_edge_feat(g1f, src2, dst, ew)

    g2 = pl.pallas_call(
        _tc2_body,
        out_shape=jax.ShapeDtypeStruct((N, D_OP), jnp.float32),
    )(s1_parts, g1f, deg_parts, b1.reshape(1, D_HID), W2p)

    s2_parts = _sc_edge_out(g2, src, dst, ew)

    out2 = pl.pallas_call(
        _tc3_body,
        out_shape=jax.ShapeDtypeStruct((N, D_OP), jnp.float32),
    )(s2_parts, g2, deg_parts, b2p.reshape(1, D_OP))

    y48 = _sc_target_gather(out2, tgt_pad)

    loss_arr, y = pl.pallas_call(
        _tc4_body,
        out_shape=[
            jax.ShapeDtypeStruct((1, 1), jnp.float32),
            jax.ShapeDtypeStruct((N_TGT, D_OUT), jnp.float32),
        ],
    )(y48, target.astype(jnp.int32).reshape(N_TGT, 1))

    return (loss_arr[0, 0], y)


# speedup vs baseline: 1.0912x; 1.0534x over previous
import functools

import jax
import jax.numpy as jnp
from jax import lax
from jax.experimental import pallas as pl
from jax.experimental.pallas import tpu as pltpu
from jax.experimental.pallas import tpu_sc as plsc

N = 10000
E = 320000
D_IN = 128
D_HID = 128
D_OUT = 40
D_OP = 48
DH2 = 64
N_TGT = 1000
N_TGT_PAD = 1024

NC = 2
NS = 16
NW = NC * NS
N_PAD = 10240
E_PER_TILE = E // NW
EB = 40
NB = E_PER_TILE // EB
EROWS = E // EB

NB_T = EROWS // NS
CH = 20
NCHUNK = NB_T // CH
NRING = 5
J4C = CH // NRING

_MESH = plsc.VectorSubcoreMesh(core_axis_name="c", subcore_axis_name="s")
_NO_TC_TILING = pltpu.CompilerParams(use_tc_tiling_on_sc=False,
                                     needs_layout_passes=False)


def _zero_rows(buf, nrows, ncol16):
    z = jnp.zeros((16,), jnp.float32)
    for i in range(nrows):
        for c in range(ncol16):
            buf[i, pl.ds(c * 16, 16)] = z


def _ew_vec(sb_ew, idxs, j16):
    cols = jnp.minimum(lax.iota(jnp.int32, 16) + j16 * 16, EB - 1)
    full = [jnp.full((16,), i, jnp.int32) for i in idxs]
    return plsc.load_gather(sb_ew, full + [cols])


@functools.partial(
    pl.kernel,
    out_type=jax.ShapeDtypeStruct((NC, N_PAD), jnp.float32),
    mesh=_MESH,
    compiler_params=_NO_TC_TILING,
    scratch_types=[
        pltpu.VMEM((NB, EB), jnp.int32),
        pltpu.VMEM((NB, EB), jnp.float32),
        pltpu.VMEM((640,), jnp.float32),
        pltpu.VMEM_SHARED((N_PAD,), jnp.float32),
        pltpu.SemaphoreType.DMA,
        pltpu.SemaphoreType.DMA,
    ],
)
def _sc_degree(dst_hbm, ew_hbm, out_hbm, sb_dst, sb_ew, zbuf, acc, ss0, ss1):
    cid = lax.axis_index("c")
    sid = lax.axis_index("s")
    tid = cid * NS + sid

    pltpu.sync_copy(dst_hbm.at[pl.ds(tid * NB, NB)], sb_dst)
    pltpu.sync_copy(ew_hbm.at[pl.ds(tid * NB, NB)], sb_ew)
    z = jnp.zeros((16,), jnp.float32)
    for i in range(40):
        zbuf[pl.ds(i * 16, 16)] = z
    pltpu.sync_copy(zbuf, acc.at[pl.ds(sid * 640, 640)])
    plsc.subcore_barrier()

    def body(j2, _):
        for par, ss in ((0, ss0), (1, ss1)):
            j = j2 * 2 + par

            @pl.when(j2 >= 1)
            def _():
                pltpu.make_async_copy(sb_ew.at[j], acc.at[sb_dst.at[j]],
                                      ss).wait()
            pltpu.async_copy(sb_ew.at[j], acc.at[sb_dst.at[j]], ss, add=True)
        return 0
    lax.fori_loop(0, NB // 2, body, 0)
    pltpu.make_async_copy(sb_ew.at[0], acc.at[sb_dst.at[0]], ss0).wait()
    pltpu.make_async_copy(sb_ew.at[0], acc.at[sb_dst.at[0]], ss1).wait()
    plsc.subcore_barrier()
    pltpu.sync_copy(acc.at[pl.ds(sid * 640, 640)],
                    out_hbm.at[cid, pl.ds(sid * 640, 640)])


@functools.partial(
    pl.kernel,
    out_type=jax.ShapeDtypeStruct((NC, N_PAD, DH2), jnp.float32),
    mesh=_MESH,
    compiler_params=_NO_TC_TILING,
    scratch_types=[
        pltpu.VMEM((2, CH, EB), jnp.int32),
        pltpu.VMEM((2, CH, EB), jnp.int32),
        pltpu.VMEM((2, CH, EB), jnp.float32),
    ] + [pltpu.VMEM((EB, DH2), jnp.float32)] * (2 * NRING)
      + [pltpu.SemaphoreType.DMA] * (2 * NRING + 1)
      + [pltpu.VMEM_SHARED((N_PAD, DH2), jnp.float32)],
)
def _sc_edge_feat(gf_hbm, src_hbm, dst_hbm, ew_hbm, out_hbm,
                  sb_src, sb_dst, sb_ew, *rest):
    grows = rest[0:NRING]
    srows = rest[NRING:2 * NRING]
    sgs = rest[2 * NRING:3 * NRING]
    sss = rest[3 * NRING:4 * NRING]
    sst = rest[4 * NRING]
    acc = rest[4 * NRING + 1]
    cid = lax.axis_index("c")
    sid = lax.axis_index("s")
    row0 = sid * NB_T

    def stage(chunk, p, sem):
        r = row0 + chunk * CH
        pltpu.async_copy(src_hbm.at[cid, pl.ds(r, CH)], sb_src.at[p], sem)
        pltpu.async_copy(dst_hbm.at[pl.ds(r, CH)], sb_dst.at[p], sem)
        pltpu.async_copy(ew_hbm.at[pl.ds(r, CH)], sb_ew.at[p], sem)

    def stage_wait(sem):
        pltpu.make_async_copy(src_hbm.at[cid, pl.ds(row0, CH)],
                              sb_src.at[0], sem).wait()
        pltpu.make_async_copy(dst_hbm.at[pl.ds(row0, CH)],
                              sb_dst.at[0], sem).wait()
        pltpu.make_async_copy(ew_hbm.at[pl.ds(row0, CH)],
                              sb_ew.at[0], sem).wait()

    stage(0, 0, sst)
    _zero_rows(srows[0], EB, DH2 // 16)
    for q in range(640 // EB):
        pltpu.sync_copy(srows[0], acc.at[pl.ds(sid * 640 + q * EB, EB)])
    stage_wait(sst)
    plsc.subcore_barrier()

    def gidx(j):
        return (j // CH) % 2, j % CH

    def scale(j, grow, srow):
        p, jw = gidx(j)
        for j16 in range(3):
            ewv = _ew_vec(sb_ew, (p, jw), j16)
            for l in range(16):
                r = j16 * 16 + l
                if r >= EB:
                    break
                s = ewv[l]
                for c in range(DH2 // 16):
                    srow[r, pl.ds(c * 16, 16)] = (
                        grow[r, pl.ds(c * 16, 16)] * s)

    def g_issue(j, grow, sg):
        p, jw = gidx(j)
        pltpu.async_copy(gf_hbm.at[sb_src.at[p, jw]], grow, sg)

    def g_wait(grow, sg):
        pltpu.make_async_copy(gf_hbm.at[sb_src.at[0, 0]], grow, sg).wait()

    def s_issue(j, srow, ss):
        p, jw = gidx(j)
        pltpu.async_copy(srow, acc.at[sb_dst.at[p, jw]], ss, add=True)

    def s_wait(srow, ss):
        pltpu.make_async_copy(srow, acc.at[sb_dst.at[0, 0]], ss).wait()

    for par in range(NRING):
        g_issue(par, grows[par], sgs[par])

    def body(j4, _):
        for par in range(NRING):
            grow, srow, sg, ss = grows[par], srows[par], sgs[par], sss[par]
            j = j4 * NRING + par
            g_wait(grow, sg)

            @pl.when(j4 >= 1)
            def _():
                s_wait(srow, ss)
            scale(j, grow, srow)
            s_issue(j, srow, ss)

            @pl.when(j4 < NB_T // NRING - 1)
            def _():
                if par == 0:
                    @pl.when(j4 % J4C == J4C - 1)
                    def _():
                        stage_wait(sst)
                g_issue(j + NRING, grow, sg)

            if par == NRING - 1:
                @pl.when((j4 % J4C == 0) & (j4 < (NCHUNK - 1) * J4C))
                def _():
                    stage(j4 // J4C + 1, (j4 // J4C + 1) % 2, sst)
        return 0
    lax.fori_loop(0, NB_T // NRING, body, 0)
    for par in range(NRING):
        s_wait(srows[par], sss[par])
    plsc.subcore_barrier()
    pltpu.sync_copy(acc.at[pl.ds(sid * 640, 640)],
                    out_hbm.at[cid, pl.ds(sid * 640, 640)])


@functools.partial(
    pl.kernel,
    out_type=jax.ShapeDtypeStruct((NC, N_PAD, D_OP), jnp.float32),
    mesh=_MESH,
    compiler_params=_NO_TC_TILING,
    scratch_types=[
        pltpu.VMEM((NB, EB), jnp.int32),
        pltpu.VMEM((NB, EB), jnp.int32),
        pltpu.VMEM((NB, EB), jnp.float32),
    ] + [pltpu.VMEM((EB, D_OP), jnp.float32)] * (2 * NRING)
      + [pltpu.SemaphoreType.DMA] * (2 * NRING)
      + [pltpu.VMEM_SHARED((N_PAD, D_OP), jnp.float32)],
)
def _sc_edge_out(g_hbm, src_hbm, dst_hbm, ew_hbm, out_hbm,
                 sb_src, sb_dst, sb_ew, *rest):
    grows = rest[0:NRING]
    srows = rest[NRING:2 * NRING]
    sgs = rest[2 * NRING:3 * NRING]
    sss = rest[3 * NRING:4 * NRING]
    acc = rest[4 * NRING]
    cid = lax.axis_index("c")
    sid = lax.axis_index("s")
    tid = cid * NS + sid

    pltpu.sync_copy(src_hbm.at[pl.ds(tid * NB, NB)], sb_src)
    pltpu.sync_copy(dst_hbm.at[pl.ds(tid * NB, NB)], sb_dst)
    pltpu.sync_copy(ew_hbm.at[pl.ds(tid * NB, NB)], sb_ew)

    _zero_rows(srows[0], EB, D_OP // 16)
    for q in range(640 // EB):
        pltpu.sync_copy(srows[0], acc.at[pl.ds(sid * 640 + q * EB, EB)])
    plsc.subcore_barrier()

    def scale(j, grow, srow):
        for j16 in range(3):
            ewv = _ew_vec(sb_ew, (j,), j16)
            for l in range(16):
                r = j16 * 16 + l
                if r >= EB:
                    break
                s = ewv[l]
                for c in range(D_OP // 16):
                    srow[r, pl.ds(c * 16, 16)] = (
                        grow[r, pl.ds(c * 16, 16)] * s)

    for par in range(NRING):
        pltpu.async_copy(g_hbm.at[sb_src.at[par]], grows[par], sgs[par])

    def body(j4, _):
        for par in range(NRING):
            grow, srow, sg, ss = grows[par], srows[par], sgs[par], sss[par]
            j = j4 * NRING + par
            pltpu.make_async_copy(g_hbm.at[sb_src.at[0]], grow, sg).wait()

            @pl.when(j4 >= 1)
            def _():
                pltpu.make_async_copy(srow, acc.at[sb_dst.at[0]], ss).wait()
            scale(j, grow, srow)
            pltpu.async_copy(srow, acc.at[sb_dst.at[j]], ss, add=True)

            @pl.when(j + NRING < NB)
            def _():
                pltpu.async_copy(g_hbm.at[sb_src.at[j + NRING]], grow, sg)
        return 0
    lax.fori_loop(0, NB // NRING, body, 0)
    for par in range(NB - (NB // NRING) * NRING):
        j = (NB // NRING) * NRING + par
        grow, srow, sg, ss = grows[par], srows[par], sgs[par], sss[par]
        pltpu.make_async_copy(g_hbm.at[sb_src.at[0]], grow, sg).wait()
        pltpu.make_async_copy(srow, acc.at[sb_dst.at[0]], ss).wait()
        scale(j, grow, srow)
        pltpu.async_copy(srow, acc.at[sb_dst.at[j]], ss, add=True)
    for par in range(NRING):
        pltpu.make_async_copy(srows[par], acc.at[sb_dst.at[0]], sss[par]).wait()
    plsc.subcore_barrier()
    pltpu.sync_copy(acc.at[pl.ds(sid * 640, 640)],
                    out_hbm.at[cid, pl.ds(sid * 640, 640)])


_TPW = N_TGT_PAD // NW


@functools.partial(
    pl.kernel,
    out_type=jax.ShapeDtypeStruct((N_TGT_PAD, D_OP), jnp.float32),
    mesh=_MESH,
    compiler_params=_NO_TC_TILING,
    scratch_types=[
        pltpu.VMEM((_TPW,), jnp.int32),
        pltpu.VMEM((_TPW, D_OP), jnp.float32),
        pltpu.SemaphoreType.DMA,
    ],
)
def _sc_target_gather(out2_hbm, tgt_hbm, y_hbm, idx_t, rows_t, sem):
    cid = lax.axis_index("c")
    sid = lax.axis_index("s")
    base = (cid * NS + sid) * _TPW
    b = pl.multiple_of(base, 8)
    pltpu.sync_copy(tgt_hbm.at[pl.ds(b, _TPW)], idx_t)
    pltpu.async_copy(out2_hbm.at[idx_t], rows_t, sem).wait()
    pltpu.sync_copy(rows_t, y_hbm.at[pl.ds(b, _TPW)])


def _tc1a_body(x_ref, w1_ref, h1_ref):
    h1_ref[...] = jnp.dot(x_ref[...], w1_ref[...],
                          preferred_element_type=jnp.float32)


def _tc1b_body(h1_ref, deg_ref, g1f_ref):
    deg = deg_ref[0, :N] + deg_ref[1, :N] + 1.0
    dinv = lax.rsqrt(deg)
    g1 = dinv[:, None] * h1_ref[...]
    g1f_ref[:N, :] = g1[:, :DH2]
    g1f_ref[N:, :] = g1[:, DH2:]


def _tc2_body(s1_ref, g1f_ref, deg_ref, b1_ref, w2_ref, g2_ref):
    deg = deg_ref[0, :N] + deg_ref[1, :N] + 1.0
    dinv = lax.rsqrt(deg)
    s1 = jnp.concatenate([s1_ref[0, :N, :], s1_ref[1, :N, :]], axis=1)
    g1 = jnp.concatenate([g1f_ref[:N, :], g1f_ref[N:, :]], axis=1)
    out1 = dinv[:, None] * (s1 + g1) + b1_ref[...]
    h2 = jnp.maximum(out1, 0.0)
    f2 = jnp.dot(h2, w2_ref[...], preferred_element_type=jnp.float32)
    g2_ref[...] = dinv[:, None] * f2


def _tc3_body(s2_ref, g2_ref, deg_ref, b2_ref, out2_ref):
    deg = deg_ref[0, :N] + deg_ref[1, :N] + 1.0
    dinv = lax.rsqrt(deg)
    s2 = s2_ref[0, :N, :] + s2_ref[1, :N, :]
    out2_ref[...] = dinv[:, None] * (s2 + g2_ref[...]) + b2_ref[...]


def _tc4_body(y48_ref, tgt_ref, loss_ref, y_ref):
    yv = y48_ref[:N_TGT, :D_OUT]
    m = jnp.max(yv, axis=1, keepdims=True)
    ex = jnp.exp(yv - m)
    lse = m[:, 0] + jnp.log(jnp.sum(ex, axis=1))
    cls = lax.broadcasted_iota(jnp.int32, (N_TGT, D_OUT), 1)
    picked = jnp.sum(jnp.where(cls == tgt_ref[...], yv, 0.0), axis=1)
    loss_ref[...] = jnp.mean(lse - picked).reshape(1, 1)
    y_ref[...] = yv


def kernel(x, edge_index, edge_weight, target_x, target, W1, b1, W2, b2):
    src = edge_index[0].astype(jnp.int32).reshape(EROWS, EB)
    dst = edge_index[1].astype(jnp.int32).reshape(EROWS, EB)
    ew = edge_weight.astype(jnp.float32).reshape(EROWS, EB)
    src2 = jnp.stack([src, src + N])
    tgt_pad = jnp.concatenate(
        [target_x.astype(jnp.int32),
         jnp.zeros((N_TGT_PAD - N_TGT,), jnp.int32)])
    W2p = jnp.pad(W2, ((0, 0), (0, D_OP - D_OUT)))
    b2p = jnp.pad(b2, (0, D_OP - D_OUT))

    deg_parts = _sc_degree(dst, ew)

    h1 = pl.pallas_call(
        _tc1a_body,
        out_shape=jax.ShapeDtypeStruct((N, D_HID), jnp.float32),
    )(x, W1)

    g1f = pl.pallas_call(
        _tc1b_body,
        out_shape=jax.ShapeDtypeStruct((2 * N, DH2), jnp.float32),
    )(h1, deg_parts)

    s1_parts = _sc_edge_feat(g1f, src2, dst, ew)

    g2 = pl.pallas_call(
        _tc2_body,
        out_shape=jax.ShapeDtypeStruct((N, D_OP), jnp.float32),
    )(s1_parts, g1f, deg_parts, b1.reshape(1, D_HID), W2p)

    s2_parts = _sc_edge_out(g2, src, dst, ew)

    out2 = pl.pallas_call(
        _tc3_body,
        out_shape=jax.ShapeDtypeStruct((N, D_OP), jnp.float32),
    )(s2_parts, g2, deg_parts, b2p.reshape(1, D_OP))

    y48 = _sc_target_gather(out2, tgt_pad)

    loss_arr, y = pl.pallas_call(
        _tc4_body,
        out_shape=[
            jax.ShapeDtypeStruct((1, 1), jnp.float32),
            jax.ShapeDtypeStruct((N_TGT, D_OUT), jnp.float32),
        ],
    )(y48, target.astype(jnp.int32).reshape(N_TGT, 1))

    return (loss_arr[0, 0], y)


# speedup vs baseline: 1.1200x; 1.0263x over previous
import functools

import jax
import jax.numpy as jnp
from jax import lax
from jax.experimental import pallas as pl
from jax.experimental.pallas import tpu as pltpu
from jax.experimental.pallas import tpu_sc as plsc

N = 10000
E = 320000
D_IN = 128
D_HID = 128
D_OUT = 40
D_OP = 48
DH2 = 64
N_TGT = 1000
N_TGT_PAD = 1024

NC = 2
NS = 16
NW = NC * NS
N_PAD = 10240
E_PER_TILE = E // NW
EB = 40
NB = E_PER_TILE // EB
EROWS = E // EB

NB_T = EROWS // NS
CH = 20
NCHUNK = NB_T // CH
NRING = 5
J4C = CH // NRING

_MESH = plsc.VectorSubcoreMesh(core_axis_name="c", subcore_axis_name="s")
_NO_TC_TILING = pltpu.CompilerParams(use_tc_tiling_on_sc=False,
                                     needs_layout_passes=False)


def _zero_rows(buf, nrows, ncol16):
    z = jnp.zeros((16,), jnp.float32)
    for i in range(nrows):
        for c in range(ncol16):
            buf[i, pl.ds(c * 16, 16)] = z


def _ew_vec(sb_ew, idxs, j16):
    cols = jnp.minimum(lax.iota(jnp.int32, 16) + j16 * 16, EB - 1)
    full = [jnp.full((16,), i, jnp.int32) for i in idxs]
    return plsc.load_gather(sb_ew, full + [cols])


@functools.partial(
    pl.kernel,
    out_type=jax.ShapeDtypeStruct((NC, N_PAD), jnp.float32),
    mesh=_MESH,
    compiler_params=_NO_TC_TILING,
    scratch_types=[
        pltpu.VMEM((NB, EB), jnp.int32),
        pltpu.VMEM((NB, EB), jnp.float32),
        pltpu.VMEM((640,), jnp.float32),
        pltpu.VMEM_SHARED((N_PAD,), jnp.float32),
        pltpu.SemaphoreType.DMA,
        pltpu.SemaphoreType.DMA,
    ],
)
def _sc_degree(dst_hbm, ew_hbm, out_hbm, sb_dst, sb_ew, zbuf, acc, ss0, ss1):
    cid = lax.axis_index("c")
    sid = lax.axis_index("s")
    tid = cid * NS + sid

    pltpu.sync_copy(dst_hbm.at[pl.ds(tid * NB, NB)], sb_dst)
    pltpu.sync_copy(ew_hbm.at[pl.ds(tid * NB, NB)], sb_ew)
    z = jnp.zeros((16,), jnp.float32)
    for i in range(40):
        zbuf[pl.ds(i * 16, 16)] = z
    pltpu.sync_copy(zbuf, acc.at[pl.ds(sid * 640, 640)])
    plsc.subcore_barrier()

    def body(j2, _):
        for par, ss in ((0, ss0), (1, ss1)):
            j = j2 * 2 + par

            @pl.when(j2 >= 1)
            def _():
                pltpu.make_async_copy(sb_ew.at[j], acc.at[sb_dst.at[j]],
                                      ss).wait()
            pltpu.async_copy(sb_ew.at[j], acc.at[sb_dst.at[j]], ss, add=True)
        return 0
    lax.fori_loop(0, NB // 2, body, 0)
    pltpu.make_async_copy(sb_ew.at[0], acc.at[sb_dst.at[0]], ss0).wait()
    pltpu.make_async_copy(sb_ew.at[0], acc.at[sb_dst.at[0]], ss1).wait()
    plsc.subcore_barrier()
    pltpu.sync_copy(acc.at[pl.ds(sid * 640, 640)],
                    out_hbm.at[cid, pl.ds(sid * 640, 640)])


@functools.partial(
    pl.kernel,
    out_type=jax.ShapeDtypeStruct((NC, N_PAD, DH2), jnp.float32),
    mesh=_MESH,
    compiler_params=_NO_TC_TILING,
    scratch_types=[
        pltpu.VMEM((2, CH, EB), jnp.int32),
        pltpu.VMEM((2, CH, EB), jnp.int32),
        pltpu.VMEM((2, CH, EB), jnp.float32),
    ] + [pltpu.VMEM((EB, DH2), jnp.bfloat16)] * NRING
      + [pltpu.VMEM((EB, DH2), jnp.float32)] * NRING
      + [pltpu.SemaphoreType.DMA] * (2 * NRING + 1)
      + [pltpu.VMEM_SHARED((N_PAD, DH2), jnp.float32)],
)
def _sc_edge_feat(gf_hbm, src_hbm, dst_hbm, ew_hbm, out_hbm,
                  sb_src, sb_dst, sb_ew, *rest):
    grows = rest[0:NRING]
    srows = rest[NRING:2 * NRING]
    sgs = rest[2 * NRING:3 * NRING]
    sss = rest[3 * NRING:4 * NRING]
    sst = rest[4 * NRING]
    acc = rest[4 * NRING + 1]
    cid = lax.axis_index("c")
    sid = lax.axis_index("s")
    row0 = sid * NB_T

    def stage(chunk, p, sem):
        r = row0 + chunk * CH
        pltpu.async_copy(src_hbm.at[cid, pl.ds(r, CH)], sb_src.at[p], sem)
        pltpu.async_copy(dst_hbm.at[pl.ds(r, CH)], sb_dst.at[p], sem)
        pltpu.async_copy(ew_hbm.at[pl.ds(r, CH)], sb_ew.at[p], sem)

    def stage_wait(sem):
        pltpu.make_async_copy(src_hbm.at[cid, pl.ds(row0, CH)],
                              sb_src.at[0], sem).wait()
        pltpu.make_async_copy(dst_hbm.at[pl.ds(row0, CH)],
                              sb_dst.at[0], sem).wait()
        pltpu.make_async_copy(ew_hbm.at[pl.ds(row0, CH)],
                              sb_ew.at[0], sem).wait()

    stage(0, 0, sst)
    _zero_rows(srows[0], EB, DH2 // 16)
    for q in range(640 // EB):
        pltpu.sync_copy(srows[0], acc.at[pl.ds(sid * 640 + q * EB, EB)])
    stage_wait(sst)
    plsc.subcore_barrier()

    def gidx(j):
        return (j // CH) % 2, j % CH

    def scale(j, grow, srow):
        p, jw = gidx(j)
        for j16 in range(3):
            ewv = _ew_vec(sb_ew, (p, jw), j16)
            for l in range(16):
                r = j16 * 16 + l
                if r >= EB:
                    break
                s = ewv[l]
                for c in range(DH2 // 32):
                    v = grow[r, pl.ds(c * 32, 32)]
                    a, b = plsc.unpack(v, format=plsc.PackFormat.INTERLEAVED)
                    srow[r, pl.ds(c * 32, 16)] = a * s
                    srow[r, pl.ds(c * 32 + 16, 16)] = b * s

    def g_issue(j, grow, sg):
        p, jw = gidx(j)
        pltpu.async_copy(gf_hbm.at[sb_src.at[p, jw]], grow, sg)

    def g_wait(grow, sg):
        pltpu.make_async_copy(gf_hbm.at[sb_src.at[0, 0]], grow, sg).wait()

    def s_issue(j, srow, ss):
        p, jw = gidx(j)
        pltpu.async_copy(srow, acc.at[sb_dst.at[p, jw]], ss, add=True)

    def s_wait(srow, ss):
        pltpu.make_async_copy(srow, acc.at[sb_dst.at[0, 0]], ss).wait()

    for par in range(NRING):
        g_issue(par, grows[par], sgs[par])

    def body(j4, _):
        for par in range(NRING):
            grow, srow, sg, ss = grows[par], srows[par], sgs[par], sss[par]
            j = j4 * NRING + par
            g_wait(grow, sg)

            @pl.when(j4 >= 1)
            def _():
                s_wait(srow, ss)
            scale(j, grow, srow)
            s_issue(j, srow, ss)

            @pl.when(j4 < NB_T // NRING - 1)
            def _():
                if par == 0:
                    @pl.when(j4 % J4C == J4C - 1)
                    def _():
                        stage_wait(sst)
                g_issue(j + NRING, grow, sg)

            if par == NRING - 1:
                @pl.when((j4 % J4C == 0) & (j4 < (NCHUNK - 1) * J4C))
                def _():
                    stage(j4 // J4C + 1, (j4 // J4C + 1) % 2, sst)
        return 0
    lax.fori_loop(0, NB_T // NRING, body, 0)
    for par in range(NRING):
        s_wait(srows[par], sss[par])
    plsc.subcore_barrier()
    pltpu.sync_copy(acc.at[pl.ds(sid * 640, 640)],
                    out_hbm.at[cid, pl.ds(sid * 640, 640)])


@functools.partial(
    pl.kernel,
    out_type=jax.ShapeDtypeStruct((NC, N_PAD, D_OP), jnp.float32),
    mesh=_MESH,
    compiler_params=_NO_TC_TILING,
    scratch_types=[
        pltpu.VMEM((NB, EB), jnp.int32),
        pltpu.VMEM((NB, EB), jnp.int32),
        pltpu.VMEM((NB, EB), jnp.float32),
    ] + [pltpu.VMEM((EB, D_OP), jnp.float32)] * (2 * NRING)
      + [pltpu.SemaphoreType.DMA] * (2 * NRING)
      + [pltpu.VMEM_SHARED((N_PAD, D_OP), jnp.float32)],
)
def _sc_edge_out(g_hbm, src_hbm, dst_hbm, ew_hbm, out_hbm,
                 sb_src, sb_dst, sb_ew, *rest):
    grows = rest[0:NRING]
    srows = rest[NRING:2 * NRING]
    sgs = rest[2 * NRING:3 * NRING]
    sss = rest[3 * NRING:4 * NRING]
    acc = rest[4 * NRING]
    cid = lax.axis_index("c")
    sid = lax.axis_index("s")
    tid = cid * NS + sid

    pltpu.sync_copy(src_hbm.at[pl.ds(tid * NB, NB)], sb_src)
    pltpu.sync_copy(dst_hbm.at[pl.ds(tid * NB, NB)], sb_dst)
    pltpu.sync_copy(ew_hbm.at[pl.ds(tid * NB, NB)], sb_ew)

    _zero_rows(srows[0], EB, D_OP // 16)
    for q in range(640 // EB):
        pltpu.sync_copy(srows[0], acc.at[pl.ds(sid * 640 + q * EB, EB)])
    plsc.subcore_barrier()

    def scale(j, grow, srow):
        for j16 in range(3):
            ewv = _ew_vec(sb_ew, (j,), j16)
            for l in range(16):
                r = j16 * 16 + l
                if r >= EB:
                    break
                s = ewv[l]
                for c in range(D_OP // 16):
                    srow[r, pl.ds(c * 16, 16)] = (
                        grow[r, pl.ds(c * 16, 16)] * s)

    for par in range(NRING):
        pltpu.async_copy(g_hbm.at[sb_src.at[par]], grows[par], sgs[par])

    def body(j4, _):
        for par in range(NRING):
            grow, srow, sg, ss = grows[par], srows[par], sgs[par], sss[par]
            j = j4 * NRING + par
            pltpu.make_async_copy(g_hbm.at[sb_src.at[0]], grow, sg).wait()

            @pl.when(j4 >= 1)
            def _():
                pltpu.make_async_copy(srow, acc.at[sb_dst.at[0]], ss).wait()
            scale(j, grow, srow)
            pltpu.async_copy(srow, acc.at[sb_dst.at[j]], ss, add=True)

            @pl.when(j + NRING < NB)
            def _():
                pltpu.async_copy(g_hbm.at[sb_src.at[j + NRING]], grow, sg)
        return 0
    lax.fori_loop(0, NB // NRING, body, 0)
    for par in range(NB - (NB // NRING) * NRING):
        j = (NB // NRING) * NRING + par
        grow, srow, sg, ss = grows[par], srows[par], sgs[par], sss[par]
        pltpu.make_async_copy(g_hbm.at[sb_src.at[0]], grow, sg).wait()
        pltpu.make_async_copy(srow, acc.at[sb_dst.at[0]], ss).wait()
        scale(j, grow, srow)
        pltpu.async_copy(srow, acc.at[sb_dst.at[j]], ss, add=True)
    for par in range(NRING):
        pltpu.make_async_copy(srows[par], acc.at[sb_dst.at[0]], sss[par]).wait()
    plsc.subcore_barrier()
    pltpu.sync_copy(acc.at[pl.ds(sid * 640, 640)],
                    out_hbm.at[cid, pl.ds(sid * 640, 640)])


_TPW = N_TGT_PAD // NW


@functools.partial(
    pl.kernel,
    out_type=jax.ShapeDtypeStruct((N_TGT_PAD, D_OP), jnp.float32),
    mesh=_MESH,
    compiler_params=_NO_TC_TILING,
    scratch_types=[
        pltpu.VMEM((_TPW,), jnp.int32),
        pltpu.VMEM((_TPW, D_OP), jnp.float32),
        pltpu.SemaphoreType.DMA,
    ],
)
def _sc_target_gather(out2_hbm, tgt_hbm, y_hbm, idx_t, rows_t, sem):
    cid = lax.axis_index("c")
    sid = lax.axis_index("s")
    base = (cid * NS + sid) * _TPW
    b = pl.multiple_of(base, 8)
    pltpu.sync_copy(tgt_hbm.at[pl.ds(b, _TPW)], idx_t)
    pltpu.async_copy(out2_hbm.at[idx_t], rows_t, sem).wait()
    pltpu.sync_copy(rows_t, y_hbm.at[pl.ds(b, _TPW)])


def _tc1a_body(x_ref, w1_ref, w1p_ref, h1_ref, h1p_ref):
    x = x_ref[...]
    h1_ref[...] = jnp.dot(x, w1_ref[...], preferred_element_type=jnp.float32)
    h1p_ref[...] = jnp.dot(x, w1p_ref[...],
                           preferred_element_type=jnp.float32)


def _tc1b_body(h1p_ref, deg_ref, g1f_ref):
    deg = deg_ref[0, :N] + deg_ref[1, :N] + 1.0
    dinv = lax.rsqrt(deg)
    g1i = (dinv[:, None] * h1p_ref[...]).astype(jnp.bfloat16)
    g1f_ref[:N, :] = g1i[:, :DH2]
    g1f_ref[N:, :] = g1i[:, DH2:]


def _tc2_body(s1_ref, h1_ref, deg_ref, b1_ref, w2_ref, g2_ref):
    deg = deg_ref[0, :N] + deg_ref[1, :N] + 1.0
    dinv = lax.rsqrt(deg)
    s1 = jnp.concatenate([s1_ref[0, :N, :], s1_ref[1, :N, :]], axis=1)
    g1 = dinv[:, None] * h1_ref[...]
    out1 = dinv[:, None] * (s1 + g1) + b1_ref[...]
    h2 = jnp.maximum(out1, 0.0)
    f2 = jnp.dot(h2, w2_ref[...], preferred_element_type=jnp.float32)
    g2_ref[...] = dinv[:, None] * f2


def _tc3_body(s2_ref, g2_ref, deg_ref, b2_ref, out2_ref):
    deg = deg_ref[0, :N] + deg_ref[1, :N] + 1.0
    dinv = lax.rsqrt(deg)
    s2 = s2_ref[0, :N, :] + s2_ref[1, :N, :]
    out2_ref[...] = dinv[:, None] * (s2 + g2_ref[...]) + b2_ref[...]


def _tc4_body(y48_ref, tgt_ref, loss_ref, y_ref):
    yv = y48_ref[:N_TGT, :D_OUT]
    m = jnp.max(yv, axis=1, keepdims=True)
    ex = jnp.exp(yv - m)
    lse = m[:, 0] + jnp.log(jnp.sum(ex, axis=1))
    cls = lax.broadcasted_iota(jnp.int32, (N_TGT, D_OUT), 1)
    picked = jnp.sum(jnp.where(cls == tgt_ref[...], yv, 0.0), axis=1)
    loss_ref[...] = jnp.mean(lse - picked).reshape(1, 1)
    y_ref[...] = yv


def kernel(x, edge_index, edge_weight, target_x, target, W1, b1, W2, b2):
    src = edge_index[0].astype(jnp.int32).reshape(EROWS, EB)
    dst = edge_index[1].astype(jnp.int32).reshape(EROWS, EB)
    ew = edge_weight.astype(jnp.float32).reshape(EROWS, EB)
    src2 = jnp.stack([src, src + N])
    tgt_pad = jnp.concatenate(
        [target_x.astype(jnp.int32),
         jnp.zeros((N_TGT_PAD - N_TGT,), jnp.int32)])
    W2p = jnp.pad(W2, ((0, 0), (0, D_OP - D_OUT)))
    b2p = jnp.pad(b2, (0, D_OP - D_OUT))

    deg_parts = _sc_degree(dst, ew)

    perm = jnp.array(
        [32 * b + 16 * j + k for b in range(4) for k in range(16)
         for j in range(2)], dtype=jnp.int32).reshape(4, 16, 2)
    perm = perm.reshape(D_HID)
    W1p = W1[:, perm]

    h1, h1p = pl.pallas_call(
        _tc1a_body,
        out_shape=[jax.ShapeDtypeStruct((N, D_HID), jnp.float32),
                   jax.ShapeDtypeStruct((N, D_HID), jnp.float32)],
    )(x, W1, W1p)

    g1f = pl.pallas_call(
        _tc1b_body,
        out_shape=jax.ShapeDtypeStruct((2 * N, DH2), jnp.bfloat16),
    )(h1p, deg_parts)

    s1_parts = _sc_edge_feat(g1f, src2, dst, ew)

    g2 = pl.pallas_call(
        _tc2_body,
        out_shape=jax.ShapeDtypeStruct((N, D_OP), jnp.float32),
    )(s1_parts, h1, deg_parts, b1.reshape(1, D_HID), W2p)

    s2_parts = _sc_edge_out(g2, src, dst, ew)

    out2 = pl.pallas_call(
        _tc3_body,
        out_shape=jax.ShapeDtypeStruct((N, D_OP), jnp.float32),
    )(s2_parts, g2, deg_parts, b2p.reshape(1, D_OP))

    y48 = _sc_target_gather(out2, tgt_pad)

    loss_arr, y = pl.pallas_call(
        _tc4_body,
        out_shape=[
            jax.ShapeDtypeStruct((1, 1), jnp.float32),
            jax.ShapeDtypeStruct((N_TGT, D_OUT), jnp.float32),
        ],
    )(y48, target.astype(jnp.int32).reshape(N_TGT, 1))

    return (loss_arr[0, 0], y)
